# baseline (speedup 1.0000x reference)
import functools

import numpy as np
import jax
import jax.numpy as jnp
from jax import lax
from jax.experimental import pallas as pl
from jax.experimental.pallas import tpu as pltpu
from jax.experimental.pallas import tpu_sc as plsc

N = 4096
NUM_FERNS = 16
K = 16
D_OUT = 32
TABLE = 2 ** K
REP = 8
RW = NUM_FERNS * REP

NC = 2
NS = 16
NW = NC * NS
SPW = N // NW
FPC = 4
NCH = NUM_FERNS // FPC
JQ = FPC * D_OUT
CHUNK = SPW * JQ
IBUF = 3
DBUF = 3
_JOBS = [(0, 0, SPW // 2), (0, SPW // 2, SPW),
         (1, 0, SPW), (2, 0, SPW),
         (3, 0, SPW // 2), (3, SPW // 2, SPW)]

_pack_np = np.zeros((NUM_FERNS * K, RW), np.float32)
for _m in range(NUM_FERNS):
    _pack_np[_m * K:(_m + 1) * K, _m * REP:(_m + 1) * REP] = (
        2.0 ** np.arange(K - 1, -1, -1)
    )[:, None]
_PACK_W = _pack_np

_IDX_BLOCKS = 4


def _idx_body(b_ref, w_ref, o_ref):
    acc = jnp.dot(b_ref[...], w_ref[...], preferred_element_type=jnp.float32)
    o_ref[...] = acc.astype(jnp.int32)


def _compute_codes(B2, nh):
    blk = nh // _IDX_BLOCKS
    return pl.pallas_call(
        _idx_body,
        out_shape=jax.ShapeDtypeStruct((nh, RW), jnp.int32),
        grid=(_IDX_BLOCKS,),
        in_specs=[
            pl.BlockSpec((blk, NUM_FERNS * K), lambda i: (i, 0)),
            pl.BlockSpec((NUM_FERNS * K, RW), lambda i: (0, 0)),
        ],
        out_specs=pl.BlockSpec((blk, RW), lambda i: (i, 0)),
    )(B2, _PACK_W)


@functools.cache
def _build_sc_gather_reduce(nh):
    mesh = plsc.VectorSubcoreMesh(core_axis_name="c", subcore_axis_name="s")
    spw = nh // NW
    chunk = spw * JQ
    jobs = [(0, 0, spw // 2), (0, spw // 2, spw),
            (1, 0, spw), (2, 0, spw),
            (3, 0, spw // 2), (3, spw // 2, spw)]

    @functools.partial(
        pl.kernel,
        out_type=jax.ShapeDtypeStruct((nh, D_OUT), jnp.float32),
        mesh=mesh,
        scratch_types=[
            pltpu.VMEM((spw * RW,), jnp.int32),
            pltpu.VMEM((IBUF, chunk), jnp.int32),
            pltpu.VMEM((DBUF, chunk), jnp.float32),
            pltpu.VMEM((spw, D_OUT), jnp.float32),
            pltpu.VMEM((D_OUT,), jnp.float32),
            pltpu.SemaphoreType.DMA,
            pltpu.SemaphoreType.DMA,
        ],
        compiler_params=pltpu.CompilerParams(use_tc_tiling_on_sc=False,
                                             needs_layout_passes=False),
    )
    def _sc_gather_reduce(codes_hbm, table_hbm, bias_hbm, out_hbm,
                          r_v, idx_v, dst_v, out_v, bias_v, sem, isem):
        wid = lax.axis_index("s") * NC + lax.axis_index("c")
        rcopy = pltpu.async_copy(
            codes_hbm.at[pl.ds(wid * spw * RW, spw * RW)], r_v, isem)
        pltpu.sync_copy(bias_hbm, bias_v)

        def init(n, carry):
            out_v[n, pl.ds(0, 16)] = bias_v[pl.ds(0, 16)]
            out_v[n, pl.ds(16, 16)] = bias_v[pl.ds(16, 16)]
            return carry

        lax.fori_loop(0, spw, init, 0)
        rcopy.wait()

        d16 = lax.iota(jnp.int32, 16)
        cbase = (d16 >> 3) * 524288 + (d16 & 7) * 128

        def build(j, buf):
            q, i0, i1 = jobs[j]

            def bld(i, carry):
                for mm in range(FPC):
                    m = q * FPC + mm
                    rv = plsc.load_gather(
                        r_v, [jnp.full((16,), i * RW + m * REP, jnp.int32)])
                    lo = (cbase + m * 2097152
                          + ((rv >> 7) * 1024 + (rv & 127)))
                    idx_v[buf, pl.ds((i - i0) * JQ + mm * D_OUT, 16)] = lo
                    idx_v[buf, pl.ds((i - i0) * JQ + mm * D_OUT + 16, 16)] = (
                        lo + 2 * 524288)
                return carry

            lax.fori_loop(i0, i1, bld, 0)

        def fire(j, buf, dbuf):
            q, i0, i1 = jobs[j]
            size = (i1 - i0) * JQ
            return pltpu.async_copy(
                table_hbm.at[idx_v.at[buf, pl.ds(0, size)]],
                dst_v.at[dbuf, pl.ds(0, size)], sem)

        njobs = len(jobs)
        copies = []
        for p in range(2):
            build(p, p)
            copies.append(fire(p, p, p))

        for j in range(njobs):
            if j + 2 < njobs:
                build(j + 2, (j + 2) % IBUF)
                copies.append(fire(j + 2, (j + 2) % IBUF, (j + 2) % DBUF))
            copies[j].wait()

            def red(i, carry, j=j):
                i0 = jobs[j][1]
                dbuf = j % DBUF
                lo = out_v[i, pl.ds(0, 16)]
                hi = out_v[i, pl.ds(16, 16)]
                for mm in range(FPC):
                    off = (i - i0) * JQ + mm * D_OUT
                    lo = lo + dst_v[dbuf, pl.ds(off, 16)]
                    hi = hi + dst_v[dbuf, pl.ds(off + 16, 16)]
                out_v[i, pl.ds(0, 16)] = lo
                out_v[i, pl.ds(16, 16)] = hi
                return carry

            lax.fori_loop(jobs[j][1], jobs[j][2], red, 0)

        pltpu.sync_copy(out_v, out_hbm.at[pl.ds(wid * spw, spw)])

    return _sc_gather_reduce


_HALVES = 2


def kernel(B, weights, bias):
    B2 = B.reshape(N, NUM_FERNS * K)
    table = (weights.reshape(NUM_FERNS, 512, 128, 4, 8)
             .transpose(0, 3, 1, 4, 2)
             .reshape(NUM_FERNS * D_OUT * TABLE))
    nh = N // _HALVES
    sc = _build_sc_gather_reduce(nh)
    outs = []
    for h in range(_HALVES):
        codes = _compute_codes(B2[h * nh:(h + 1) * nh], nh)
        outs.append(sc(codes.reshape(nh * RW), table, bias))
    return jnp.concatenate(outs, axis=0)


# speedup vs baseline: 1.0918x; 1.0918x over previous
import functools

import numpy as np
import jax
import jax.numpy as jnp
from jax import lax
from jax.experimental import pallas as pl
from jax.experimental.pallas import tpu as pltpu
from jax.experimental.pallas import tpu_sc as plsc

N = 4096
NUM_FERNS = 16
K = 16
D_OUT = 32
TABLE = 2 ** K
REP = 8
RW = NUM_FERNS * REP

NC = 2
NS = 16
NW = NC * NS
SPW = N // NW
FPC = 4
NCH = NUM_FERNS // FPC
JQ = FPC * D_OUT
CHUNK = SPW * JQ
IBUF = 3
DBUF = 3
_JOBS = [(0, 0, SPW // 2), (0, SPW // 2, SPW),
         (1, 0, SPW), (2, 0, SPW),
         (3, 0, SPW // 2), (3, SPW // 2, SPW)]

_pack_np = np.zeros((NUM_FERNS * K, RW), np.float32)
for _m in range(NUM_FERNS):
    _pack_np[_m * K:(_m + 1) * K, _m * REP:(_m + 1) * REP] = (
        2.0 ** np.arange(K - 1, -1, -1)
    )[:, None]
_PACK_W = _pack_np

_IDX_BLOCKS = 4


def _idx_body(b_ref, w_ref, o_ref):
    acc = jnp.dot(b_ref[...], w_ref[...], preferred_element_type=jnp.float32)
    o_ref[...] = acc.astype(jnp.int32)


def _compute_codes(B2):
    blk = N // _IDX_BLOCKS
    return pl.pallas_call(
        _idx_body,
        out_shape=jax.ShapeDtypeStruct((N, RW), jnp.int32),
        grid=(_IDX_BLOCKS,),
        in_specs=[
            pl.BlockSpec((blk, NUM_FERNS * K), lambda i: (i, 0)),
            pl.BlockSpec((NUM_FERNS * K, RW), lambda i: (0, 0)),
        ],
        out_specs=pl.BlockSpec((blk, RW), lambda i: (i, 0)),
    )(B2, _PACK_W)


@functools.cache
def _build_sc_gather_reduce():
    mesh = plsc.VectorSubcoreMesh(core_axis_name="c", subcore_axis_name="s")

    @functools.partial(
        pl.kernel,
        out_type=jax.ShapeDtypeStruct((N, D_OUT), jnp.float32),
        mesh=mesh,
        scratch_types=[
            pltpu.VMEM((SPW * RW,), jnp.int32),
            pltpu.VMEM((IBUF, CHUNK), jnp.int32),
            pltpu.VMEM((DBUF, CHUNK), jnp.float32),
            pltpu.VMEM((SPW, D_OUT), jnp.float32),
            pltpu.VMEM((D_OUT,), jnp.float32),
            pltpu.SemaphoreType.DMA,
            pltpu.SemaphoreType.DMA,
        ],
        compiler_params=pltpu.CompilerParams(use_tc_tiling_on_sc=False,
                                             needs_layout_passes=False),
    )
    def _sc_gather_reduce(codes_hbm, table_hbm, bias_hbm, out_hbm,
                          r_v, idx_v, dst_v, out_v, bias_v, sem, isem):
        wid = lax.axis_index("s") * NC + lax.axis_index("c")
        rcopy = pltpu.async_copy(
            codes_hbm.at[pl.ds(wid * SPW * RW, SPW * RW)], r_v, isem)
        pltpu.sync_copy(bias_hbm, bias_v)

        def init(n, carry):
            out_v[n, pl.ds(0, 16)] = bias_v[pl.ds(0, 16)]
            out_v[n, pl.ds(16, 16)] = bias_v[pl.ds(16, 16)]
            return carry

        lax.fori_loop(0, SPW, init, 0)
        rcopy.wait()

        d16 = lax.iota(jnp.int32, 16)
        cbase = (d16 >> 3) * 524288 + (d16 & 7) * 128

        def build(j, buf):
            q, i0, i1 = _JOBS[j]

            def bld(i, carry):
                for mm in range(FPC):
                    m = q * FPC + mm
                    rv = plsc.load_gather(
                        r_v, [jnp.full((16,), i * RW + m * REP, jnp.int32)])
                    lo = (cbase + m * 2097152
                          + ((rv >> 7) * 1024 + (rv & 127)))
                    idx_v[buf, pl.ds((i - i0) * JQ + mm * D_OUT, 16)] = lo
                    idx_v[buf, pl.ds((i - i0) * JQ + mm * D_OUT + 16, 16)] = (
                        lo + 2 * 524288)
                return carry

            lax.fori_loop(i0, i1, bld, 0)

        def fire(j, buf, dbuf):
            q, i0, i1 = _JOBS[j]
            size = (i1 - i0) * JQ
            return pltpu.async_copy(
                table_hbm.at[idx_v.at[buf, pl.ds(0, size)]],
                dst_v.at[dbuf, pl.ds(0, size)], sem)

        njobs = len(_JOBS)
        copies = []
        for p in range(2):
            build(p, p)
            copies.append(fire(p, p, p))

        for j in range(njobs):
            if j + 2 < njobs:
                build(j + 2, (j + 2) % IBUF)
                copies.append(fire(j + 2, (j + 2) % IBUF, (j + 2) % DBUF))
            copies[j].wait()

            def red(i, carry, j=j):
                i0 = _JOBS[j][1]
                dbuf = j % DBUF
                lo = out_v[i, pl.ds(0, 16)]
                hi = out_v[i, pl.ds(16, 16)]
                for mm in range(FPC):
                    off = (i - i0) * JQ + mm * D_OUT
                    lo = lo + dst_v[dbuf, pl.ds(off, 16)]
                    hi = hi + dst_v[dbuf, pl.ds(off + 16, 16)]
                out_v[i, pl.ds(0, 16)] = lo
                out_v[i, pl.ds(16, 16)] = hi
                return carry

            lax.fori_loop(_JOBS[j][1], _JOBS[j][2], red, 0)

        pltpu.sync_copy(out_v, out_hbm.at[pl.ds(wid * SPW, SPW)])

    return _sc_gather_reduce


def kernel(B, weights, bias):
    B2 = B.reshape(N, NUM_FERNS * K)
    codes = _compute_codes(B2)
    table = (weights.reshape(NUM_FERNS, 512, 128, 4, 8)
             .transpose(0, 3, 1, 4, 2)
             .reshape(NUM_FERNS * D_OUT * TABLE))
    return _build_sc_gather_reduce()(codes.reshape(N * RW), table, bias)
